# baseline (speedup 1.0000x reference)
import numpy as np
import jax
import jax.numpy as jnp
from jax import lax
from jax.experimental import pallas as pl
from jax.experimental.pallas import tpu as pltpu

_KSIZE = 5
_VARIANCE = 1.0


def _gauss_taps():
    n = np.arange(_KSIZE, dtype=np.float64) - (_KSIZE - 1) / 2.0
    g = np.exp(-0.5 * (n / float(_VARIANCE)) ** 2)
    return tuple(float(v) for v in g)


def _band(n: int, taps, lo: int, transposed: bool) -> np.ndarray:
    K = len(taps)
    m = np.zeros((n, n), np.float64)
    for j in range(K):
        k = (j - lo) if transposed else (lo - j)
        if abs(k) < n:
            m += np.diag(np.full(n - abs(k), taps[j], np.float64), k=k)
    return m.astype(np.float32)


def _blur_kernel(x_ref, th_ref, tw_ref, o_ref):
    tw = tw_ref[...]
    th = th_ref[...]
    nb, H, W = x_ref.shape

    xb = x_ref[...].reshape(nb * H, W).astype(jnp.bfloat16)
    mid = jnp.dot(xb, tw, preferred_element_type=jnp.float32)
    mid = mid.astype(jnp.bfloat16).reshape(nb, H, W)
    for b in range(nb):
        o_ref[b] = jnp.dot(th, mid[b],
                           preferred_element_type=jnp.float32
                           ).astype(o_ref.dtype)


def kernel(x):
    N, C, H, W = x.shape
    assert C == 1
    lo = (_KSIZE - 1) // 2
    taps = _gauss_taps()
    x3 = x[:, 0]

    Wk = W if W % 128 == 0 else ((W + 127) // 128) * 128
    if Wk != W:
        x3 = jnp.pad(x3, ((0, 0), (0, 0), (0, Wk - W)))

    t_w = jnp.asarray(_band(Wk, taps, lo, transposed=False), jnp.bfloat16)
    t_h = jnp.asarray(_band(H, taps, lo, transposed=True), jnp.bfloat16)

    b_blk = min(4, N)
    out = pl.pallas_call(
        _blur_kernel,
        out_shape=jax.ShapeDtypeStruct((N, H, Wk), x.dtype),
        grid=(pl.cdiv(N, b_blk),),
        in_specs=[
            pl.BlockSpec((b_blk, H, Wk), lambda b: (b, 0, 0)),
            pl.BlockSpec((H, H), lambda b: (0, 0)),
            pl.BlockSpec((Wk, Wk), lambda b: (0, 0)),
        ],
        out_specs=pl.BlockSpec((b_blk, H, Wk), lambda b: (b, 0, 0)),
        compiler_params=pltpu.CompilerParams(
            dimension_semantics=("parallel",),
            vmem_limit_bytes=64 * 1024 * 1024,
        ),
    )(x3, t_h, t_w)

    if Wk != W:
        out = out[:, :, :W]
    return out[:, None, :, :]


# speedup vs baseline: 1.2117x; 1.2117x over previous
import numpy as np
import jax
import jax.numpy as jnp
from jax import lax
from jax.experimental import pallas as pl
from jax.experimental.pallas import tpu as pltpu

_KSIZE = 5
_VARIANCE = 1.0


def _gauss_taps():
    n = np.arange(_KSIZE, dtype=np.float64) - (_KSIZE - 1) / 2.0
    g = np.exp(-0.5 * (n / float(_VARIANCE)) ** 2)
    return tuple(float(v) for v in g)


def _band(n: int, taps, lo: int, transposed: bool) -> np.ndarray:
    K = len(taps)
    m = np.zeros((n, n), np.float64)
    for j in range(K):
        k = (j - lo) if transposed else (lo - j)
        if abs(k) < n:
            m += np.diag(np.full(n - abs(k), taps[j], np.float64), k=k)
    return m.astype(np.float32)


def _blur_kernel(x_ref, th_ref, tw_ref, o_ref):
    tw = tw_ref[...]
    th = th_ref[...]
    nb, H, W = x_ref.shape

    xb = x_ref[...].reshape(nb * H, W)
    mid = jnp.dot(xb, tw, preferred_element_type=jnp.float32)
    mid = mid.reshape(nb, H, W)
    for b in range(nb):
        o_ref[b] = jnp.dot(th, mid[b],
                           preferred_element_type=jnp.float32
                           ).astype(o_ref.dtype)


def kernel(x):
    N, C, H, W = x.shape
    assert C == 1
    lo = (_KSIZE - 1) // 2
    taps = _gauss_taps()
    x3 = x[:, 0]

    Wk = W if W % 128 == 0 else ((W + 127) // 128) * 128
    if Wk != W:
        x3 = jnp.pad(x3, ((0, 0), (0, 0), (0, Wk - W)))

    t_w = jnp.asarray(_band(Wk, taps, lo, transposed=False), jnp.float32)
    t_h = jnp.asarray(_band(H, taps, lo, transposed=True), jnp.float32)

    b_blk = min(8, N)
    out = pl.pallas_call(
        _blur_kernel,
        out_shape=jax.ShapeDtypeStruct((N, H, Wk), x.dtype),
        grid=(pl.cdiv(N, b_blk),),
        in_specs=[
            pl.BlockSpec((b_blk, H, Wk), lambda b: (b, 0, 0)),
            pl.BlockSpec((H, H), lambda b: (0, 0)),
            pl.BlockSpec((Wk, Wk), lambda b: (0, 0)),
        ],
        out_specs=pl.BlockSpec((b_blk, H, Wk), lambda b: (b, 0, 0)),
        compiler_params=pltpu.CompilerParams(
            dimension_semantics=("parallel",),
            vmem_limit_bytes=64 * 1024 * 1024,
        ),
    )(x3, t_h, t_w)

    if Wk != W:
        out = out[:, :, :W]
    return out[:, None, :, :]


# speedup vs baseline: 1.2965x; 1.0699x over previous
import numpy as np
import jax
import jax.numpy as jnp
from jax import lax
from jax.experimental import pallas as pl
from jax.experimental.pallas import tpu as pltpu

_KSIZE = 5
_VARIANCE = 1.0


def _gauss_taps():
    n = np.arange(_KSIZE, dtype=np.float64) - (_KSIZE - 1) / 2.0
    g = np.exp(-0.5 * (n / float(_VARIANCE)) ** 2)
    return tuple(float(v) for v in g)


def _band(n: int, taps, lo: int, transposed: bool) -> np.ndarray:
    K = len(taps)
    m = np.zeros((n, n), np.float64)
    for j in range(K):
        k = (j - lo) if transposed else (lo - j)
        if abs(k) < n:
            m += np.diag(np.full(n - abs(k), taps[j], np.float64), k=k)
    return m.astype(np.float32)


def _blur_kernel(x_ref, th_ref, tw_ref, o_ref):
    tw = tw_ref[...]
    th = th_ref[...]
    nb, H, W = x_ref.shape

    xb = x_ref[...].reshape(nb * H, W)
    mid = jnp.dot(xb, tw, preferred_element_type=jnp.float32)
    mid = mid.reshape(nb, H, W)
    for b in range(nb):
        o_ref[b] = jnp.dot(th, mid[b],
                           preferred_element_type=jnp.float32
                           ).astype(o_ref.dtype)


def kernel(x):
    N, C, H, W = x.shape
    assert C == 1
    lo = (_KSIZE - 1) // 2
    taps = _gauss_taps()
    x3 = x[:, 0]

    Wk = W if W % 128 == 0 else ((W + 127) // 128) * 128
    if Wk != W:
        x3 = jnp.pad(x3, ((0, 0), (0, 0), (0, Wk - W)))

    t_w = jnp.asarray(_band(Wk, taps, lo, transposed=False), jnp.float32)
    t_h = jnp.asarray(_band(H, taps, lo, transposed=True), jnp.float32)

    b_blk = min(16, N)
    out = pl.pallas_call(
        _blur_kernel,
        out_shape=jax.ShapeDtypeStruct((N, H, Wk), x.dtype),
        grid=(pl.cdiv(N, b_blk),),
        in_specs=[
            pl.BlockSpec((b_blk, H, Wk), lambda b: (b, 0, 0)),
            pl.BlockSpec((H, H), lambda b: (0, 0)),
            pl.BlockSpec((Wk, Wk), lambda b: (0, 0)),
        ],
        out_specs=pl.BlockSpec((b_blk, H, Wk), lambda b: (b, 0, 0)),
        compiler_params=pltpu.CompilerParams(
            dimension_semantics=("parallel",),
            vmem_limit_bytes=64 * 1024 * 1024,
        ),
    )(x3, t_h, t_w)

    if Wk != W:
        out = out[:, :, :W]
    return out[:, None, :, :]
